# baseline (speedup 1.0000x reference)
import functools

import jax
import jax.numpy as jnp
from jax import lax
from jax.experimental import pallas as pl
from jax.experimental.pallas import tpu as pltpu
from jax.experimental.pallas import tpu_sc as plsc

N = 10000
NPAD = 10256
E = 320000
D = 128
K = 128


@functools.lru_cache(maxsize=None)
def _build_sc_agg():
    info = plsc.get_sparse_core_info()
    nc, ns = info.num_cores, info.num_subcores
    nw = nc * ns
    e_per_w = E // nw
    ch = -(-e_per_w // K)
    ch += ch % 2
    e_pad = ch * K
    assert e_per_w * nw == E
    rps = (NPAD // ns) & ~7
    tail = NPAD - ns * rps

    mesh = plsc.VectorSubcoreMesh(core_axis_name="c", subcore_axis_name="s")

    @functools.partial(
        pl.kernel,
        mesh=mesh,
        out_type=jax.ShapeDtypeStruct((nc, NPAD, D), jnp.float32),
        scratch_types=[
            pltpu.VMEM((ch, K), jnp.int32),
            pltpu.VMEM((2, K), jnp.int32),
            pltpu.VMEM((K, D), jnp.float32),
            pltpu.VMEM((K, D), jnp.float32),
            pltpu.SemaphoreType.DMA,
            pltpu.SemaphoreType.DMA,
            pltpu.SemaphoreType.DMA,
            pltpu.SemaphoreType.DMA,
            pltpu.VMEM_SHARED((NPAD, D), jnp.float32),
        ],
    )
    def sc_agg(h_hbm, src_hbm, dst_hbm, zeros_hbm, out_hbm,
               src_v, dst_s, b0, b1, sem0, sem1, semd0, semd1, acc_shared):
        cid = lax.axis_index("c")
        sid = lax.axis_index("s")
        wid = sid * nc + cid
        base = wid * e_pad

        def dst_row(j):
            return dst_hbm.at[pl.ds(base + j * K, K)]

        pltpu.sync_copy(zeros_hbm.at[pl.ds(sid * rps, rps)],
                        acc_shared.at[pl.ds(sid * rps, rps)])

        @pl.when(sid == ns - 1)
        def _():
            pltpu.sync_copy(zeros_hbm.at[pl.ds(ns * rps, tail)],
                            acc_shared.at[pl.ds(ns * rps, tail)])

        pltpu.sync_copy(src_hbm.at[wid], src_v)
        pltpu.sync_copy(dst_row(0), dst_s.at[0])
        plsc.subcore_barrier()

        pltpu.async_copy(h_hbm.at[src_v.at[0]], b0, sem0)
        pltpu.async_copy(dst_row(1), dst_s.at[1], semd1)

        def body(j2, carry):
            j = 2 * j2
            pltpu.async_copy(h_hbm.at[src_v.at[j + 1]], b1, sem1)

            @pl.when(j2 > 0)
            def _():
                pltpu.make_async_copy(dst_row(0), dst_s.at[0], semd0).wait()

            pltpu.make_async_copy(h_hbm.at[src_v.at[0]], b0, sem0).wait()
            pltpu.sync_copy(b0, acc_shared.at[dst_s.at[0]], add=True)

            @pl.when(j + 2 < ch)
            def _():
                pltpu.async_copy(dst_row(j + 2), dst_s.at[0], semd0)
                pltpu.async_copy(h_hbm.at[src_v.at[j + 2]], b0, sem0)

            pltpu.make_async_copy(dst_row(0), dst_s.at[1], semd1).wait()
            pltpu.make_async_copy(h_hbm.at[src_v.at[0]], b1, sem1).wait()
            pltpu.sync_copy(b1, acc_shared.at[dst_s.at[1]], add=True)

            @pl.when(j + 3 < ch)
            def _():
                pltpu.async_copy(dst_row(j + 3), dst_s.at[1], semd1)

            return carry

        lax.fori_loop(0, ch // 2, body, 0)
        plsc.subcore_barrier()

        pltpu.sync_copy(acc_shared.at[pl.ds(sid * rps, rps)],
                        out_hbm.at[cid, pl.ds(sid * rps, rps)])

        @pl.when(sid == ns - 1)
        def _():
            pltpu.sync_copy(acc_shared.at[pl.ds(ns * rps, tail)],
                            out_hbm.at[cid, pl.ds(ns * rps, tail)])

    return sc_agg, nc, nw, ch, e_per_w


def _mlp(h, agg, eps, W1, b1, W2, b2, nc):
    blk = 2000

    def body(h_ref, a_ref, eps_ref, w1_ref, b1_ref, w2_ref, b2_ref, o_ref):
        z = (1.0 + eps_ref[0, 0]) * h_ref[...]
        for c in range(nc):
            z = z + a_ref[c]
        z = jnp.maximum(
            jnp.dot(z, w1_ref[...], preferred_element_type=jnp.float32)
            + b1_ref[...], 0.0)
        z = jnp.dot(z, w2_ref[...], preferred_element_type=jnp.float32) + b2_ref[...]
        o_ref[...] = jnp.maximum(z, 0.0)

    return pl.pallas_call(
        body,
        grid=(N // blk,),
        in_specs=[
            pl.BlockSpec((blk, D), lambda i: (i, 0)),
            pl.BlockSpec((nc, blk, D), lambda i: (0, i, 0)),
            pl.BlockSpec((1, 1), lambda i: (0, 0)),
            pl.BlockSpec((D, D), lambda i: (0, 0)),
            pl.BlockSpec((1, D), lambda i: (0, 0)),
            pl.BlockSpec((D, D), lambda i: (0, 0)),
            pl.BlockSpec((1, D), lambda i: (0, 0)),
        ],
        out_specs=pl.BlockSpec((blk, D), lambda i: (i, 0)),
        out_shape=jax.ShapeDtypeStruct((N, D), jnp.float32),
    )(h, agg, eps.reshape(1, 1), W1, b1.reshape(1, D), W2, b2.reshape(1, D))


def kernel(x, edge_index, eps0, W1_0, b1_0, W2_0, b2_0,
           eps1, W1_1, b1_1, W2_1, b2_1):
    sc_agg, nc, nw, ch, e_per_w = _build_sc_agg()
    pad = ch * K - e_per_w
    src = jnp.pad(edge_index[0].reshape(nw, e_per_w),
                  ((0, 0), (0, pad))).reshape(nw, ch, K)
    trash = (N + 8 * jnp.arange(nw, dtype=jnp.int32)[:, None]
             + jnp.arange(pad, dtype=jnp.int32)[None, :] % 8)
    dst = jnp.concatenate(
        [edge_index[1].reshape(nw, e_per_w), trash], axis=1).reshape(-1)
    zeros = jnp.zeros((NPAD, D), jnp.float32)

    agg0 = sc_agg(x, src, dst, zeros)
    h = _mlp(x, agg0, eps0, W1_0, b1_0, W2_0, b2_0, nc)
    agg1 = sc_agg(h, src, dst, zeros)
    h = _mlp(h, agg1, eps1, W1_1, b1_1, W2_1, b2_1, nc)
    return h


# speedup vs baseline: 1.2764x; 1.2764x over previous
import functools

import jax
import jax.numpy as jnp
from jax import lax
from jax.experimental import pallas as pl
from jax.experimental.pallas import tpu as pltpu
from jax.experimental.pallas import tpu_sc as plsc

N = 10000
NPAD = 10240
E = 320000
D = 128
K = 128


@functools.lru_cache(maxsize=None)
def _build_sc_agg():
    info = plsc.get_sparse_core_info()
    nc, ns = info.num_cores, info.num_subcores
    nw = nc * ns
    e_per_w = E // nw
    ch = -(-e_per_w // K)
    e_pad = ch * K
    assert e_per_w * nw == E
    rows_per_sub = NPAD // ns

    mesh = plsc.VectorSubcoreMesh(core_axis_name="c", subcore_axis_name="s")

    @functools.partial(
        pl.kernel,
        mesh=mesh,
        out_type=jax.ShapeDtypeStruct((nc, NPAD, D), jnp.float32),
        scratch_types=[
            pltpu.VMEM((ch, K), jnp.int32),
            pltpu.VMEM((ch, K), jnp.int32),
            pltpu.VMEM((K, D), jnp.float32),
            pltpu.SemaphoreType.DMA,
            pltpu.VMEM_SHARED((NPAD, D), jnp.float32),
        ],
    )
    def sc_agg(h_hbm, src_hbm, dst_hbm, zeros_hbm, out_hbm,
               src_v, dst_v, rows_v, sem, acc_shared):
        cid = lax.axis_index("c")
        sid = lax.axis_index("s")
        wid = sid * nc + cid

        pltpu.sync_copy(
            zeros_hbm.at[pl.ds(sid * rows_per_sub, rows_per_sub)],
            acc_shared.at[pl.ds(sid * rows_per_sub, rows_per_sub)],
        )
        pltpu.sync_copy(src_hbm.at[wid], src_v)
        pltpu.sync_copy(dst_hbm.at[wid], dst_v)
        plsc.subcore_barrier()

        def body(j, carry):
            pltpu.async_copy(h_hbm.at[src_v.at[j]], rows_v, sem).wait()
            pltpu.sync_copy(rows_v, acc_shared.at[dst_v.at[j]], add=True)
            return carry

        lax.fori_loop(0, ch, body, 0)
        plsc.subcore_barrier()

        pltpu.sync_copy(
            acc_shared.at[pl.ds(sid * rows_per_sub, rows_per_sub)],
            out_hbm.at[cid, pl.ds(sid * rows_per_sub, rows_per_sub)],
        )

    return sc_agg, nc, nw, ch, e_per_w


def _mlp(h, agg, eps, W1, b1, W2, b2, nc):
    blk = 2000

    def body(h_ref, a_ref, eps_ref, w1_ref, b1_ref, w2_ref, b2_ref, o_ref):
        z = (1.0 + eps_ref[0, 0]) * h_ref[...]
        for c in range(nc):
            z = z + a_ref[c]
        z = jnp.maximum(
            jnp.dot(z, w1_ref[...], preferred_element_type=jnp.float32)
            + b1_ref[...], 0.0)
        z = jnp.dot(z, w2_ref[...], preferred_element_type=jnp.float32) + b2_ref[...]
        o_ref[...] = jnp.maximum(z, 0.0)

    return pl.pallas_call(
        body,
        grid=(N // blk,),
        in_specs=[
            pl.BlockSpec((blk, D), lambda i: (i, 0)),
            pl.BlockSpec((nc, blk, D), lambda i: (0, i, 0)),
            pl.BlockSpec((1, 1), lambda i: (0, 0)),
            pl.BlockSpec((D, D), lambda i: (0, 0)),
            pl.BlockSpec((1, D), lambda i: (0, 0)),
            pl.BlockSpec((D, D), lambda i: (0, 0)),
            pl.BlockSpec((1, D), lambda i: (0, 0)),
        ],
        out_specs=pl.BlockSpec((blk, D), lambda i: (i, 0)),
        out_shape=jax.ShapeDtypeStruct((N, D), jnp.float32),
    )(h, agg, eps.reshape(1, 1), W1, b1.reshape(1, D), W2, b2.reshape(1, D))


def kernel(x, edge_index, eps0, W1_0, b1_0, W2_0, b2_0,
           eps1, W1_1, b1_1, W2_1, b2_1):
    sc_agg, nc, nw, ch, e_per_w = _build_sc_agg()
    pad = ch * K - e_per_w
    src = jnp.pad(edge_index[0].reshape(nw, e_per_w),
                  ((0, 0), (0, pad))).reshape(nw, ch, K)
    trash = (N + 8 * jnp.arange(nw, dtype=jnp.int32)[:, None]
             + jnp.arange(pad, dtype=jnp.int32)[None, :] % 8)
    dst = jnp.concatenate(
        [edge_index[1].reshape(nw, e_per_w), trash], axis=1).reshape(nw, ch, K)
    zeros = jnp.zeros((NPAD, D), jnp.float32)

    agg0 = sc_agg(x, src, dst, zeros)
    h = _mlp(x, agg0, eps0, W1_0, b1_0, W2_0, b2_0, nc)
    agg1 = sc_agg(h, src, dst, zeros)
    h = _mlp(h, agg1, eps1, W1_1, b1_1, W2_1, b2_1, nc)
    return h


# speedup vs baseline: 3.0263x; 2.3710x over previous
import functools

import jax
import jax.numpy as jnp
from jax import lax
from jax.experimental import pallas as pl
from jax.experimental.pallas import tpu as pltpu
from jax.experimental.pallas import tpu_sc as plsc

N = 10000
E = 320000
D = 128
K = 100
SH = 14
L = 16


@functools.lru_cache(maxsize=None)
def _build_sc_agg():
    info = plsc.get_sparse_core_info()
    nc, ns = info.num_cores, info.num_subcores
    nw = nc * ns
    e_per_w = E // nw
    ch = e_per_w // K
    assert ch * K == e_per_w and ch % 2 == 0
    rps = (N // ns) & ~7
    tail = N - ns * rps
    offs = list(range(0, K - L, L)) + [K - L]

    mesh = plsc.VectorSubcoreMesh(core_axis_name="c", subcore_axis_name="s")

    @functools.partial(
        pl.kernel,
        mesh=mesh,
        out_type=jax.ShapeDtypeStruct((nc, N, D), jnp.float32),
        scratch_types=[
            pltpu.VMEM((ch, K), jnp.int32),
            pltpu.VMEM((2, K), jnp.int32),
            pltpu.VMEM((2, K), jnp.int32),
            pltpu.VMEM((K, D), jnp.float32),
            pltpu.VMEM((K, D), jnp.float32),
            pltpu.SemaphoreType.DMA,
            pltpu.SemaphoreType.DMA,
            pltpu.VMEM_SHARED((N, D), jnp.float32),
        ],
    )
    def sc_agg(h_hbm, combo_hbm, zeros_hbm, out_hbm,
               combo_v, src_dec, dst_dec, b0, b1, sem0, sem1, acc_shared):
        cid = lax.axis_index("c")
        sid = lax.axis_index("s")
        wid = sid * nc + cid

        def decode(j, p):
            for o in offs:
                w = combo_v[j, pl.ds(o, L)]
                src_dec[p, pl.ds(o, L)] = w >> SH
                dst_dec[p, pl.ds(o, L)] = w & ((1 << SH) - 1)

        pltpu.sync_copy(zeros_hbm.at[pl.ds(sid * rps, rps)],
                        acc_shared.at[pl.ds(sid * rps, rps)])

        @pl.when(sid == ns - 1)
        def _():
            pltpu.sync_copy(zeros_hbm.at[pl.ds(ns * rps, tail)],
                            acc_shared.at[pl.ds(ns * rps, tail)])

        pltpu.sync_copy(combo_hbm.at[wid], combo_v)
        plsc.subcore_barrier()

        decode(0, 0)
        pltpu.async_copy(h_hbm.at[src_dec.at[0]], b0, sem0)

        def body(j2, carry):
            j = 2 * j2
            decode(j + 1, 1)
            pltpu.async_copy(h_hbm.at[src_dec.at[1]], b1, sem1)
            pltpu.make_async_copy(h_hbm.at[src_dec.at[0]], b0, sem0).wait()
            pltpu.sync_copy(b0, acc_shared.at[dst_dec.at[0]], add=True)

            @pl.when(j + 2 < ch)
            def _():
                decode(j + 2, 0)
                pltpu.async_copy(h_hbm.at[src_dec.at[0]], b0, sem0)

            pltpu.make_async_copy(h_hbm.at[src_dec.at[1]], b1, sem1).wait()
            pltpu.sync_copy(b1, acc_shared.at[dst_dec.at[1]], add=True)
            return carry

        lax.fori_loop(0, ch // 2, body, 0)
        plsc.subcore_barrier()

        pltpu.sync_copy(acc_shared.at[pl.ds(sid * rps, rps)],
                        out_hbm.at[cid, pl.ds(sid * rps, rps)])

        @pl.when(sid == ns - 1)
        def _():
            pltpu.sync_copy(acc_shared.at[pl.ds(ns * rps, tail)],
                            out_hbm.at[cid, pl.ds(ns * rps, tail)])

    return sc_agg, nc, nw, ch


def _mlp(h, agg, eps, W1, b1, W2, b2, nc):
    blk = 2000

    def body(h_ref, a_ref, eps_ref, w1_ref, b1_ref, w2_ref, b2_ref, o_ref):
        z = (1.0 + eps_ref[0, 0]) * h_ref[...]
        for c in range(nc):
            z = z + a_ref[c]
        z = jnp.maximum(
            jnp.dot(z, w1_ref[...], preferred_element_type=jnp.float32)
            + b1_ref[...], 0.0)
        z = jnp.dot(z, w2_ref[...], preferred_element_type=jnp.float32) + b2_ref[...]
        o_ref[...] = jnp.maximum(z, 0.0)

    return pl.pallas_call(
        body,
        grid=(N // blk,),
        in_specs=[
            pl.BlockSpec((blk, D), lambda i: (i, 0)),
            pl.BlockSpec((nc, blk, D), lambda i: (0, i, 0)),
            pl.BlockSpec((1, 1), lambda i: (0, 0)),
            pl.BlockSpec((D, D), lambda i: (0, 0)),
            pl.BlockSpec((1, D), lambda i: (0, 0)),
            pl.BlockSpec((D, D), lambda i: (0, 0)),
            pl.BlockSpec((1, D), lambda i: (0, 0)),
        ],
        out_specs=pl.BlockSpec((blk, D), lambda i: (i, 0)),
        out_shape=jax.ShapeDtypeStruct((N, D), jnp.float32),
    )(h, agg, eps.reshape(1, 1), W1, b1.reshape(1, D), W2, b2.reshape(1, D))


def kernel(x, edge_index, eps0, W1_0, b1_0, W2_0, b2_0,
           eps1, W1_1, b1_1, W2_1, b2_1):
    sc_agg, nc, nw, ch = _build_sc_agg()
    combo = ((edge_index[0] << SH) | edge_index[1]).reshape(nw, ch, K)
    zeros = jnp.zeros((N, D), jnp.float32)

    agg0 = sc_agg(x, combo, zeros)
    h = _mlp(x, agg0, eps0, W1_0, b1_0, W2_0, b2_0, nc)
    agg1 = sc_agg(h, combo, zeros)
    h = _mlp(h, agg1, eps1, W1_1, b1_1, W2_1, b2_1, nc)
    return h


# speedup vs baseline: 3.1437x; 1.0388x over previous
import functools

import jax
import jax.numpy as jnp
from jax import lax
from jax.experimental import pallas as pl
from jax.experimental.pallas import tpu as pltpu
from jax.experimental.pallas import tpu_sc as plsc

N = 10000
E = 320000
D = 128
K = 125
SH = 14
L = 16


@functools.lru_cache(maxsize=None)
def _build_sc_agg():
    info = plsc.get_sparse_core_info()
    nc, ns = info.num_cores, info.num_subcores
    nw = nc * ns
    e_per_w = E // nw
    ch = e_per_w // K
    assert ch * K == e_per_w and ch % 2 == 0
    rps = (N // ns) & ~7
    tail = N - ns * rps
    offs = list(range(0, K - L, L)) + [K - L]

    mesh = plsc.VectorSubcoreMesh(core_axis_name="c", subcore_axis_name="s")

    @functools.partial(
        pl.kernel,
        mesh=mesh,
        out_type=jax.ShapeDtypeStruct((nc, N, D), jnp.float32),
        scratch_types=[
            pltpu.VMEM((ch, K), jnp.int32),
            pltpu.VMEM((2, K), jnp.int32),
            pltpu.VMEM((2, K), jnp.int32),
            pltpu.VMEM((K, D), jnp.float32),
            pltpu.VMEM((K, D), jnp.float32),
            pltpu.SemaphoreType.DMA,
            pltpu.SemaphoreType.DMA,
            pltpu.VMEM_SHARED((N, D), jnp.float32),
        ],
    )
    def sc_agg(h_hbm, combo_hbm, zeros_hbm, out_hbm,
               combo_v, src_dec, dst_dec, b0, b1, sem0, sem1, acc_shared):
        cid = lax.axis_index("c")
        sid = lax.axis_index("s")
        wid = sid * nc + cid

        def decode(j, p):
            for o in offs:
                w = combo_v[j, pl.ds(o, L)]
                src_dec[p, pl.ds(o, L)] = w >> SH
                dst_dec[p, pl.ds(o, L)] = w & ((1 << SH) - 1)

        pltpu.sync_copy(zeros_hbm.at[pl.ds(sid * rps, rps)],
                        acc_shared.at[pl.ds(sid * rps, rps)])

        @pl.when(sid == ns - 1)
        def _():
            pltpu.sync_copy(zeros_hbm.at[pl.ds(ns * rps, tail)],
                            acc_shared.at[pl.ds(ns * rps, tail)])

        pltpu.sync_copy(combo_hbm.at[wid], combo_v)
        plsc.subcore_barrier()

        decode(0, 0)
        pltpu.async_copy(h_hbm.at[src_dec.at[0]], b0, sem0)

        def body(j2, carry):
            j = 2 * j2
            decode(j + 1, 1)
            pltpu.async_copy(h_hbm.at[src_dec.at[1]], b1, sem1)
            pltpu.make_async_copy(h_hbm.at[src_dec.at[0]], b0, sem0).wait()
            pltpu.sync_copy(b0, acc_shared.at[dst_dec.at[0]], add=True)

            @pl.when(j + 2 < ch)
            def _():
                decode(j + 2, 0)
                pltpu.async_copy(h_hbm.at[src_dec.at[0]], b0, sem0)

            pltpu.make_async_copy(h_hbm.at[src_dec.at[1]], b1, sem1).wait()
            pltpu.sync_copy(b1, acc_shared.at[dst_dec.at[1]], add=True)
            return carry

        lax.fori_loop(0, ch // 2, body, 0)
        plsc.subcore_barrier()

        pltpu.sync_copy(acc_shared.at[pl.ds(sid * rps, rps)],
                        out_hbm.at[cid, pl.ds(sid * rps, rps)])

        @pl.when(sid == ns - 1)
        def _():
            pltpu.sync_copy(acc_shared.at[pl.ds(ns * rps, tail)],
                            out_hbm.at[cid, pl.ds(ns * rps, tail)])

    return sc_agg, nc, nw, ch


def _mlp(h, agg, eps, W1, b1, W2, b2, nc):
    blk = 2000

    def body(h_ref, a_ref, eps_ref, w1_ref, b1_ref, w2_ref, b2_ref, o_ref):
        z = (1.0 + eps_ref[0, 0]) * h_ref[...]
        for c in range(nc):
            z = z + a_ref[c]
        z = jnp.maximum(
            jnp.dot(z, w1_ref[...], preferred_element_type=jnp.float32)
            + b1_ref[...], 0.0)
        z = jnp.dot(z, w2_ref[...], preferred_element_type=jnp.float32) + b2_ref[...]
        o_ref[...] = jnp.maximum(z, 0.0)

    return pl.pallas_call(
        body,
        grid=(N // blk,),
        in_specs=[
            pl.BlockSpec((blk, D), lambda i: (i, 0)),
            pl.BlockSpec((nc, blk, D), lambda i: (0, i, 0)),
            pl.BlockSpec((1, 1), lambda i: (0, 0)),
            pl.BlockSpec((D, D), lambda i: (0, 0)),
            pl.BlockSpec((1, D), lambda i: (0, 0)),
            pl.BlockSpec((D, D), lambda i: (0, 0)),
            pl.BlockSpec((1, D), lambda i: (0, 0)),
        ],
        out_specs=pl.BlockSpec((blk, D), lambda i: (i, 0)),
        out_shape=jax.ShapeDtypeStruct((N, D), jnp.float32),
    )(h, agg, eps.reshape(1, 1), W1, b1.reshape(1, D), W2, b2.reshape(1, D))


def kernel(x, edge_index, eps0, W1_0, b1_0, W2_0, b2_0,
           eps1, W1_1, b1_1, W2_1, b2_1):
    sc_agg, nc, nw, ch = _build_sc_agg()
    combo = ((edge_index[0] << SH) | edge_index[1]).reshape(nw, ch, K)
    zeros = jnp.zeros((N, D), jnp.float32)

    agg0 = sc_agg(x, combo, zeros)
    h = _mlp(x, agg0, eps0, W1_0, b1_0, W2_0, b2_0, nc)
    agg1 = sc_agg(h, combo, zeros)
    h = _mlp(h, agg1, eps1, W1_1, b1_1, W2_1, b2_1, nc)
    return h
